# initial kernel scaffold (unmeasured)
import jax
import jax.numpy as jnp
from jax import lax
from jax.experimental import pallas as pl
from jax.experimental.pallas import tpu as pltpu

N_DEV = 16
B_LOC = 2
SQ = 256
HQ = 64
DH = 64
DM = 512
HD = HQ * DH
H_LOC = HQ // N_DEV
CHUNK = H_LOC * DH


def kernel(x, Wq, K_ext, V_ext, Wo):
    my = lax.axis_index("i")

    xb = x.astype(jnp.bfloat16)
    wq = Wq.astype(jnp.bfloat16)
    wo = Wo.astype(jnp.bfloat16)
    k = lax.dynamic_slice_in_dim(K_ext, my * B_LOC, B_LOC, axis=0)
    v = lax.dynamic_slice_in_dim(V_ext, my * B_LOC, B_LOC, axis=0)
    k = k.transpose(0, 2, 1, 3).astype(jnp.bfloat16)
    v = v.transpose(0, 2, 1, 3).astype(jnp.bfloat16)

    def body(x_ref, wq_ref, k_ref, v_ref, wo_ref, out_ref,
             wqf, wof, q_buf, ctx_buf,
             wq_ssem, wq_rsem, wo_ssem, wo_rsem):
        me = lax.axis_index("i")
        left = lax.rem(me + N_DEV - 1, N_DEV)
        right = lax.rem(me + 1, N_DEV)

        bar = pltpu.get_barrier_semaphore()
        pl.semaphore_signal(bar, inc=1, device_id=(left,),
                            device_id_type=pl.DeviceIdType.MESH)
        pl.semaphore_signal(bar, inc=1, device_id=(right,),
                            device_id_type=pl.DeviceIdType.MESH)
        pl.semaphore_wait(bar, 2)

        wqf[:, pl.ds(me * CHUNK, CHUNK)] = wq_ref[:, :]
        wof[pl.ds(me * CHUNK, CHUNK), :] = wo_ref[:, :]

        for h in range(N_DEV - 1):
            o = lax.rem(me + 2 * N_DEV - h, N_DEV)
            wq_rdma = pltpu.make_async_remote_copy(
                src_ref=wqf.at[:, pl.ds(o * CHUNK, CHUNK)],
                dst_ref=wqf.at[:, pl.ds(o * CHUNK, CHUNK)],
                send_sem=wq_ssem.at[h],
                recv_sem=wq_rsem.at[h],
                device_id=(right,),
                device_id_type=pl.DeviceIdType.MESH,
            )
            wo_rdma = pltpu.make_async_remote_copy(
                src_ref=wof.at[pl.ds(o * CHUNK, CHUNK), :],
                dst_ref=wof.at[pl.ds(o * CHUNK, CHUNK), :],
                send_sem=wo_ssem.at[h],
                recv_sem=wo_rsem.at[h],
                device_id=(right,),
                device_id_type=pl.DeviceIdType.MESH,
            )
            wq_rdma.start()
            wo_rdma.start()
            wq_rdma.wait()
            wo_rdma.wait()

        rows = lax.broadcasted_iota(jnp.int32, (SQ, SQ), 0)
        cols = lax.broadcasted_iota(jnp.int32, (SQ, SQ), 1)
        mask = (cols // 64) <= (rows // 64)

        for b in range(B_LOC):
            q_buf[:, :] = jnp.dot(
                x_ref[b], wqf[:, :], preferred_element_type=jnp.bfloat16
            )

            def head_step(h, _):
                qh = q_buf[:, pl.ds(h * DH, DH)]
                kh = k_ref[b, h]
                s = lax.dot_general(
                    qh, kh, (((1,), (1,)), ((), ())),
                    preferred_element_type=jnp.float32,
                ) * 0.125
                s = jnp.where(mask, s, -1e9)
                m = jnp.max(s, axis=1, keepdims=True)
                w = jnp.exp(s - m)
                w = w / jnp.sum(w, axis=1, keepdims=True)
                ctx = jnp.dot(
                    w.astype(jnp.bfloat16), v_ref[b, h],
                    preferred_element_type=jnp.float32,
                )
                ctx_buf[:, pl.ds(h * DH, DH)] = ctx.astype(jnp.bfloat16)
                return 0

            lax.fori_loop(0, HQ, head_step, 0)
            out_ref[b] = jnp.dot(
                ctx_buf[:, :], wof[:, :], preferred_element_type=jnp.float32
            )

    return pl.pallas_call(
        body,
        out_shape=jax.ShapeDtypeStruct((B_LOC, SQ, DM), jnp.float32),
        in_specs=[pl.BlockSpec(memory_space=pltpu.VMEM)] * 5,
        out_specs=pl.BlockSpec(memory_space=pltpu.VMEM),
        scratch_shapes=[
            pltpu.VMEM((DM, HD), jnp.bfloat16),
            pltpu.VMEM((HD, DM), jnp.bfloat16),
            pltpu.VMEM((SQ, HD), jnp.bfloat16),
            pltpu.VMEM((SQ, HD), jnp.bfloat16),
            pltpu.SemaphoreType.DMA((N_DEV - 1,)),
            pltpu.SemaphoreType.DMA((N_DEV - 1,)),
            pltpu.SemaphoreType.DMA((N_DEV - 1,)),
            pltpu.SemaphoreType.DMA((N_DEV - 1,)),
        ],
        compiler_params=pltpu.CompilerParams(collective_id=0),
    )(xb, wq, k, v, wo)


# baseline (device time: 201388 ns/iter reference)
import jax
import jax.numpy as jnp
from jax import lax
from jax.experimental import pallas as pl
from jax.experimental.pallas import tpu as pltpu

N_DEV = 16
B_LOC = 2
SQ = 256
HQ = 64
DH = 64
DM = 512
HD = HQ * DH
H_LOC = HQ // N_DEV
CHUNK = H_LOC * DH


def kernel(x, Wq, K_ext, V_ext, Wo):
    my = lax.axis_index("i")

    xb = x.astype(jnp.bfloat16)
    wq = Wq.astype(jnp.bfloat16)
    wo = Wo.astype(jnp.bfloat16)
    k = lax.dynamic_slice_in_dim(K_ext, my * B_LOC, B_LOC, axis=0)
    v = lax.dynamic_slice_in_dim(V_ext, my * B_LOC, B_LOC, axis=0)
    k = k.transpose(0, 2, 1, 3).astype(jnp.bfloat16)
    v = v.transpose(0, 2, 1, 3).astype(jnp.bfloat16)

    def body(x_ref, wq_ref, k_ref, v_ref, wo_ref, out_ref,
             wqf, wof, q_buf, ctx_buf,
             wq_ssem, wq_rsem, wo_ssem, wo_rsem):
        me = lax.axis_index("i")
        left = lax.rem(me + N_DEV - 1, N_DEV)
        right = lax.rem(me + 1, N_DEV)

        bar = pltpu.get_barrier_semaphore()
        pl.semaphore_signal(bar, inc=1, device_id=(left,),
                            device_id_type=pl.DeviceIdType.MESH)
        pl.semaphore_signal(bar, inc=1, device_id=(right,),
                            device_id_type=pl.DeviceIdType.MESH)
        pl.semaphore_wait(bar, 2)

        wqf[:, pl.ds(me * CHUNK, CHUNK)] = wq_ref[:, :]
        wof[pl.ds(me * CHUNK, CHUNK), :] = wo_ref[:, :]

        for h in range(N_DEV - 1):
            o = lax.rem(me + 2 * N_DEV - h, N_DEV)
            wq_rdma = pltpu.make_async_remote_copy(
                src_ref=wqf.at[:, pl.ds(o * CHUNK, CHUNK)],
                dst_ref=wqf.at[:, pl.ds(o * CHUNK, CHUNK)],
                send_sem=wq_ssem.at[h],
                recv_sem=wq_rsem.at[h],
                device_id=(right,),
                device_id_type=pl.DeviceIdType.MESH,
            )
            wo_rdma = pltpu.make_async_remote_copy(
                src_ref=wof.at[pl.ds(o * CHUNK, CHUNK), :],
                dst_ref=wof.at[pl.ds(o * CHUNK, CHUNK), :],
                send_sem=wo_ssem.at[h],
                recv_sem=wo_rsem.at[h],
                device_id=(right,),
                device_id_type=pl.DeviceIdType.MESH,
            )
            wq_rdma.start()
            wo_rdma.start()
            wq_rdma.wait()
            wo_rdma.wait()

        rows = lax.broadcasted_iota(jnp.int32, (SQ, SQ), 0)
        cols = lax.broadcasted_iota(jnp.int32, (SQ, SQ), 1)
        mask = (cols // 64) <= (rows // 64)

        for b in range(B_LOC):
            q_buf[:, :] = jnp.dot(
                x_ref[b], wqf[:, :], preferred_element_type=jnp.float32
            ).astype(jnp.bfloat16)

            def head_step(j, _):
                q2 = q_buf[:, pl.ds(j * 2 * DH, 2 * DH)]
                ctxs = []
                for t in range(2):
                    h = 2 * j + t
                    qh = q2[:, t * DH:(t + 1) * DH]
                    kh = k_ref[b, h]
                    s = lax.dot_general(
                        qh, kh, (((1,), (1,)), ((), ())),
                        preferred_element_type=jnp.float32,
                    ) * 0.125
                    s = jnp.where(mask, s, -1e9)
                    m = jnp.max(s, axis=1, keepdims=True)
                    w = jnp.exp(s - m)
                    w = w / jnp.sum(w, axis=1, keepdims=True)
                    ctx = jnp.dot(
                        w.astype(jnp.bfloat16), v_ref[b, h],
                        preferred_element_type=jnp.float32,
                    )
                    ctxs.append(ctx.astype(jnp.bfloat16))
                ctx_buf[:, pl.ds(j * 2 * DH, 2 * DH)] = jnp.concatenate(
                    ctxs, axis=1
                )
                return 0

            lax.fori_loop(0, HQ // 2, head_step, 0)
            out_ref[b] = jnp.dot(
                ctx_buf[:, :], wof[:, :], preferred_element_type=jnp.float32
            )

    return pl.pallas_call(
        body,
        out_shape=jax.ShapeDtypeStruct((B_LOC, SQ, DM), jnp.float32),
        in_specs=[pl.BlockSpec(memory_space=pltpu.VMEM)] * 5,
        out_specs=pl.BlockSpec(memory_space=pltpu.VMEM),
        scratch_shapes=[
            pltpu.VMEM((DM, HD), jnp.bfloat16),
            pltpu.VMEM((HD, DM), jnp.bfloat16),
            pltpu.VMEM((SQ, HD), jnp.bfloat16),
            pltpu.VMEM((SQ, HD), jnp.bfloat16),
            pltpu.SemaphoreType.DMA((N_DEV - 1,)),
            pltpu.SemaphoreType.DMA((N_DEV - 1,)),
            pltpu.SemaphoreType.DMA((N_DEV - 1,)),
            pltpu.SemaphoreType.DMA((N_DEV - 1,)),
        ],
        compiler_params=pltpu.CompilerParams(collective_id=0),
    )(xb, wq, k, v, wo)


# device time: 107252 ns/iter; 1.8777x vs baseline; 1.8777x over previous
import jax
import jax.numpy as jnp
from jax import lax
from jax.experimental import pallas as pl
from jax.experimental.pallas import tpu as pltpu

N_DEV = 16
B_LOC = 2
SQ = 256
HQ = 64
DH = 64
DM = 512
HD = HQ * DH
H_LOC = HQ // N_DEV
CHUNK = H_LOC * DH

N_RIGHT = 8
N_LEFT = 7


def kernel(x, Wq, K_ext, V_ext, Wo):
    my = lax.axis_index("i")

    xb = x.astype(jnp.bfloat16)
    wq = Wq.astype(jnp.bfloat16)
    wo = Wo.astype(jnp.bfloat16)
    k = lax.dynamic_slice_in_dim(K_ext, my * B_LOC, B_LOC, axis=0)
    v = lax.dynamic_slice_in_dim(V_ext, my * B_LOC, B_LOC, axis=0)
    k = k.transpose(0, 2, 1, 3).astype(jnp.bfloat16)
    v = v.transpose(0, 2, 1, 3).astype(jnp.bfloat16)

    def body(x_ref, wq_ref, k_ref, v_ref, wo_ref, out_ref,
             wqf, wof, q_buf, sq_s, sq_r, so_s, so_r):
        me = lax.axis_index("i")
        left = lax.rem(me + N_DEV - 1, N_DEV)
        right = lax.rem(me + 1, N_DEV)

        bar = pltpu.get_barrier_semaphore()
        pl.semaphore_signal(bar, inc=1, device_id=(left,),
                            device_id_type=pl.DeviceIdType.MESH)
        pl.semaphore_signal(bar, inc=1, device_id=(right,),
                            device_id_type=pl.DeviceIdType.MESH)
        pl.semaphore_wait(bar, 2)

        sends = []

        def start_hop(direction, h, src_wq, src_wo, origin):
            tgt = right if direction == "r" else left
            sem = h if direction == "r" else N_RIGHT + h
            dst_q = wqf.at[:, pl.ds(origin * CHUNK, CHUNK)]
            dst_o = wof.at[pl.ds(origin * CHUNK, CHUNK), :]
            rq = pltpu.make_async_remote_copy(
                src_ref=src_wq, dst_ref=dst_q,
                send_sem=sq_s.at[sem], recv_sem=sq_r.at[sem],
                device_id=(tgt,), device_id_type=pl.DeviceIdType.MESH,
            )
            ro = pltpu.make_async_remote_copy(
                src_ref=src_wo, dst_ref=dst_o,
                send_sem=so_s.at[sem], recv_sem=so_r.at[sem],
                device_id=(tgt,), device_id_type=pl.DeviceIdType.MESH,
            )
            rq.start()
            ro.start()
            sends.append(rq)
            sends.append(ro)
            return rq, ro

        def wait_hop(direction, h, origin):
            sem = h if direction == "r" else N_RIGHT + h
            dst_q = wqf.at[:, pl.ds(origin * CHUNK, CHUNK)]
            dst_o = wof.at[pl.ds(origin * CHUNK, CHUNK), :]
            pltpu.make_async_remote_copy(
                src_ref=dst_q, dst_ref=dst_q,
                send_sem=sq_s.at[sem], recv_sem=sq_r.at[sem],
                device_id=(right,), device_id_type=pl.DeviceIdType.MESH,
            ).wait_recv()
            pltpu.make_async_remote_copy(
                src_ref=dst_o, dst_ref=dst_o,
                send_sem=so_s.at[sem], recv_sem=so_r.at[sem],
                device_id=(right,), device_id_type=pl.DeviceIdType.MESH,
            ).wait_recv()

        rows = lax.broadcasted_iota(jnp.int32, (SQ, SQ), 0)
        cols = lax.broadcasted_iota(jnp.int32, (SQ, SQ), 1)
        mask = (cols // 64) <= (rows // 64)

        def compute_chunk(o):
            def batch_step(b, _):
                q_buf[:, :] = jnp.dot(
                    x_ref[b], wqf[:, pl.ds(o * CHUNK, CHUNK)],
                    preferred_element_type=jnp.float32,
                ).astype(jnp.bfloat16)

                def pair_step(p, _):
                    q2 = q_buf[:, pl.ds(p * 2 * DH, 2 * DH)]
                    ctxs = []
                    for t in range(2):
                        h = o * H_LOC + p * 2 + t
                        qh = q2[:, t * DH:(t + 1) * DH]
                        s = lax.dot_general(
                            qh, k_ref[b, h], (((1,), (1,)), ((), ())),
                            preferred_element_type=jnp.float32,
                        ) * 0.125
                        s = jnp.where(mask, s, -1e9)
                        m = jnp.max(s, axis=1, keepdims=True)
                        w = jnp.exp(s - m)
                        w = w / jnp.sum(w, axis=1, keepdims=True)
                        ctx = jnp.dot(
                            w.astype(jnp.bfloat16), v_ref[b, h],
                            preferred_element_type=jnp.float32,
                        )
                        ctxs.append(ctx.astype(jnp.bfloat16))
                    q_buf[:, pl.ds(p * 2 * DH, 2 * DH)] = jnp.concatenate(
                        ctxs, axis=1
                    )
                    return 0

                lax.fori_loop(0, H_LOC // 2, pair_step, 0)
                out_ref[b] += jnp.dot(
                    q_buf[:, :], wof[pl.ds(o * CHUNK, CHUNK), :],
                    preferred_element_type=jnp.float32,
                )
                return 0

            lax.fori_loop(0, B_LOC, batch_step, 0)

        start_hop("r", 0, wq_ref, wo_ref, me)
        start_hop("l", 0, wq_ref, wo_ref, me)

        wqf[:, pl.ds(me * CHUNK, CHUNK)] = wq_ref[:, :]
        wof[pl.ds(me * CHUNK, CHUNK), :] = wo_ref[:, :]
        out_ref[...] = jnp.zeros((B_LOC, SQ, DM), jnp.float32)
        compute_chunk(me)

        for s in range(N_RIGHT):
            r_o = lax.rem(me + (2 * N_DEV - 1 - s), N_DEV)
            l_o = lax.rem(me + 1 + s, N_DEV)
            wait_hop("r", s, r_o)
            if s < N_LEFT:
                wait_hop("l", s, l_o)
            if s + 1 < N_RIGHT:
                start_hop("r", s + 1,
                          wqf.at[:, pl.ds(r_o * CHUNK, CHUNK)],
                          wof.at[pl.ds(r_o * CHUNK, CHUNK), :], r_o)
            if s + 1 < N_LEFT:
                start_hop("l", s + 1,
                          wqf.at[:, pl.ds(l_o * CHUNK, CHUNK)],
                          wof.at[pl.ds(l_o * CHUNK, CHUNK), :], l_o)
            compute_chunk(r_o)
            if s < N_LEFT:
                compute_chunk(l_o)

        for d in sends:
            d.wait_send()

    return pl.pallas_call(
        body,
        out_shape=jax.ShapeDtypeStruct((B_LOC, SQ, DM), jnp.float32),
        in_specs=[pl.BlockSpec(memory_space=pltpu.VMEM)] * 5,
        out_specs=pl.BlockSpec(memory_space=pltpu.VMEM),
        scratch_shapes=[
            pltpu.VMEM((DM, HD), jnp.bfloat16),
            pltpu.VMEM((HD, DM), jnp.bfloat16),
            pltpu.VMEM((SQ, CHUNK), jnp.bfloat16),
            pltpu.SemaphoreType.DMA((N_DEV - 1,)),
            pltpu.SemaphoreType.DMA((N_DEV - 1,)),
            pltpu.SemaphoreType.DMA((N_DEV - 1,)),
            pltpu.SemaphoreType.DMA((N_DEV - 1,)),
        ],
        compiler_params=pltpu.CompilerParams(collective_id=0),
    )(xb, wq, k, v, wo)


# device time: 83167 ns/iter; 2.4215x vs baseline; 1.2896x over previous
import jax
import jax.numpy as jnp
from jax import lax
from jax.experimental import pallas as pl
from jax.experimental.pallas import tpu as pltpu

N_DEV = 16
B_LOC = 2
SQ = 256
HQ = 64
DH = 64
DM = 512
HD = HQ * DH
H_LOC = HQ // N_DEV
CHUNK = H_LOC * DH
MROW = B_LOC * SQ

N_RIGHT = 8
N_LEFT = 7


def kernel(x, Wq, K_ext, V_ext, Wo):
    my = lax.axis_index("i")

    x2 = x.astype(jnp.bfloat16).reshape(MROW, DM)
    wq = Wq.astype(jnp.bfloat16)
    wo = Wo.astype(jnp.bfloat16)
    k = lax.dynamic_slice_in_dim(K_ext, my * B_LOC, B_LOC, axis=0)
    v = lax.dynamic_slice_in_dim(V_ext, my * B_LOC, B_LOC, axis=0)
    k = k.transpose(0, 2, 1, 3).astype(jnp.bfloat16)
    v = v.transpose(0, 2, 1, 3).astype(jnp.bfloat16)

    def body(x_ref, wq_ref, k_ref, v_ref, wo_ref, out_ref,
             wqf, wof, sq_s, sq_r, so_s, so_r):
        me = lax.axis_index("i")
        left = lax.rem(me + N_DEV - 1, N_DEV)
        right = lax.rem(me + 1, N_DEV)

        bar = pltpu.get_barrier_semaphore()
        pl.semaphore_signal(bar, inc=1, device_id=(left,),
                            device_id_type=pl.DeviceIdType.MESH)
        pl.semaphore_signal(bar, inc=1, device_id=(right,),
                            device_id_type=pl.DeviceIdType.MESH)
        pl.semaphore_wait(bar, 2)

        def hop_rdmas(direction, sem, src_wq, src_wo, origin):
            tgt = right if direction == "r" else left
            rq = pltpu.make_async_remote_copy(
                src_ref=src_wq,
                dst_ref=wqf.at[:, pl.ds(origin * CHUNK, CHUNK)],
                send_sem=sq_s.at[sem], recv_sem=sq_r.at[sem],
                device_id=(tgt,), device_id_type=pl.DeviceIdType.MESH,
            )
            ro = pltpu.make_async_remote_copy(
                src_ref=src_wo,
                dst_ref=wof.at[pl.ds(origin * CHUNK, CHUNK), :],
                send_sem=so_s.at[sem], recv_sem=so_r.at[sem],
                device_id=(tgt,), device_id_type=pl.DeviceIdType.MESH,
            )
            return rq, ro

        def start_hop(direction, sem, origin):
            rq, ro = hop_rdmas(
                direction, sem,
                wqf.at[:, pl.ds(origin * CHUNK, CHUNK)],
                wof.at[pl.ds(origin * CHUNK, CHUNK), :], origin)
            rq.start()
            ro.start()

        def wait_hop(direction, sem, origin):
            rq, ro = hop_rdmas(direction, sem,
                               wqf.at[:, pl.ds(origin * CHUNK, CHUNK)],
                               wof.at[pl.ds(origin * CHUNK, CHUNK), :],
                               origin)
            rq.wait_recv()
            ro.wait_recv()

        rows = lax.broadcasted_iota(jnp.int32, (SQ, SQ), 0)
        cols = lax.broadcasted_iota(jnp.int32, (SQ, SQ), 1)
        mask = (cols // 64) <= (rows // 64)

        def compute_chunk(o):
            q2 = jnp.dot(
                x_ref[:, :], wqf[:, pl.ds(o * CHUNK, CHUNK)],
                preferred_element_type=jnp.float32,
            ).astype(jnp.bfloat16)
            ctxs = []
            for b in range(B_LOC):
                hctx = []
                for t in range(H_LOC):
                    h = o * H_LOC + t
                    qh = q2[b * SQ:(b + 1) * SQ, t * DH:(t + 1) * DH]
                    s = lax.dot_general(
                        qh, k_ref[b, h], (((1,), (1,)), ((), ())),
                        preferred_element_type=jnp.float32,
                    ) * 0.125
                    s = jnp.where(mask, s, -1e9)
                    m = jnp.max(s, axis=1, keepdims=True)
                    w = jnp.exp(s - m)
                    w = w / jnp.sum(w, axis=1, keepdims=True)
                    ctx = jnp.dot(
                        w.astype(jnp.bfloat16), v_ref[b, h],
                        preferred_element_type=jnp.float32,
                    )
                    hctx.append(ctx.astype(jnp.bfloat16))
                ctxs.append(jnp.concatenate(hctx, axis=1))
            ctx2 = jnp.concatenate(ctxs, axis=0)
            out_ref[:, :] += jnp.dot(
                ctx2, wof[pl.ds(o * CHUNK, CHUNK), :],
                preferred_element_type=jnp.float32,
            )

        rq0, ro0 = hop_rdmas("r", 0, wq_ref, wo_ref, me)
        rq0.start()
        ro0.start()
        lq0, lo0 = hop_rdmas("l", N_RIGHT, wq_ref, wo_ref, me)
        lq0.start()
        lo0.start()

        wqf[:, pl.ds(me * CHUNK, CHUNK)] = wq_ref[:, :]
        wof[pl.ds(me * CHUNK, CHUNK), :] = wo_ref[:, :]
        out_ref[:, :] = jnp.zeros((MROW, DM), jnp.float32)
        compute_chunk(me)

        def step(s, _):
            r_o = lax.rem(me + 2 * N_DEV - 1 - s, N_DEV)
            l_o = lax.rem(me + 1 + s, N_DEV)
            wait_hop("r", s, r_o)

            @pl.when(s + 1 < N_RIGHT)
            def _():
                start_hop("r", s + 1, r_o)

            @pl.when(s < N_LEFT)
            def _():
                wait_hop("l", N_RIGHT + s, l_o)

            @pl.when(s + 1 < N_LEFT)
            def _():
                start_hop("l", N_RIGHT + s + 1, l_o)

            compute_chunk(r_o)

            @pl.when(s < N_LEFT)
            def _():
                compute_chunk(l_o)

            return 0

        lax.fori_loop(0, N_RIGHT, step, 0)

        for h in range(N_RIGHT):
            rq, ro = hop_rdmas("r", h, wq_ref, wo_ref, me)
            rq.wait_send()
            ro.wait_send()
        for h in range(N_LEFT):
            rq, ro = hop_rdmas("l", N_RIGHT + h, wq_ref, wo_ref, me)
            rq.wait_send()
            ro.wait_send()

    out2 = pl.pallas_call(
        body,
        out_shape=jax.ShapeDtypeStruct((MROW, DM), jnp.float32),
        in_specs=[pl.BlockSpec(memory_space=pltpu.VMEM)] * 5,
        out_specs=pl.BlockSpec(memory_space=pltpu.VMEM),
        scratch_shapes=[
            pltpu.VMEM((DM, HD), jnp.bfloat16),
            pltpu.VMEM((HD, DM), jnp.bfloat16),
            pltpu.SemaphoreType.DMA((N_DEV - 1,)),
            pltpu.SemaphoreType.DMA((N_DEV - 1,)),
            pltpu.SemaphoreType.DMA((N_DEV - 1,)),
            pltpu.SemaphoreType.DMA((N_DEV - 1,)),
        ],
        compiler_params=pltpu.CompilerParams(collective_id=0),
    )(x2, wq, k, v, wo)
    return out2.reshape(B_LOC, SQ, DM)


# device time: 78741 ns/iter; 2.5576x vs baseline; 1.0562x over previous
import jax
import jax.numpy as jnp
from jax import lax
from jax.experimental import pallas as pl
from jax.experimental.pallas import tpu as pltpu

N_DEV = 16
B_LOC = 2
SQ = 256
HQ = 64
DH = 64
DM = 512
HD = HQ * DH
H_LOC = HQ // N_DEV
CHUNK = H_LOC * DH
MROW = B_LOC * SQ

N_RIGHT = 8
N_LEFT = 7


def kernel(x, Wq, K_ext, V_ext, Wo):
    my = lax.axis_index("i")

    x2 = x.astype(jnp.bfloat16).reshape(MROW, DM)
    wq = Wq.astype(jnp.bfloat16)
    wo = Wo.astype(jnp.bfloat16)
    k = lax.dynamic_slice_in_dim(K_ext, my * B_LOC, B_LOC, axis=0)
    v = lax.dynamic_slice_in_dim(V_ext, my * B_LOC, B_LOC, axis=0)
    k = k.transpose(0, 2, 1, 3).astype(jnp.bfloat16)
    v = v.transpose(0, 2, 1, 3).astype(jnp.bfloat16)

    def body(x_ref, wq_ref, k_ref, v_ref, wo_ref, out_ref,
             wqf, wof, sq_s, sq_r, so_s, so_r):
        me = lax.axis_index("i")
        left = lax.rem(me + N_DEV - 1, N_DEV)
        right = lax.rem(me + 1, N_DEV)

        bar = pltpu.get_barrier_semaphore()
        pl.semaphore_signal(bar, inc=1, device_id=(left,),
                            device_id_type=pl.DeviceIdType.MESH)
        pl.semaphore_signal(bar, inc=1, device_id=(right,),
                            device_id_type=pl.DeviceIdType.MESH)
        pl.semaphore_wait(bar, 2)

        def q_rdma(direction, sem, src, origin):
            tgt = right if direction == "r" else left
            return pltpu.make_async_remote_copy(
                src_ref=src,
                dst_ref=wqf.at[:, pl.ds(origin * CHUNK, CHUNK)],
                send_sem=sq_s.at[sem], recv_sem=sq_r.at[sem],
                device_id=(tgt,), device_id_type=pl.DeviceIdType.MESH,
            )

        def o_rdma(direction, sem, src, origin):
            tgt = right if direction == "r" else left
            return pltpu.make_async_remote_copy(
                src_ref=src,
                dst_ref=wof.at[pl.ds(origin * CHUNK, CHUNK), :],
                send_sem=so_s.at[sem], recv_sem=so_r.at[sem],
                device_id=(tgt,), device_id_type=pl.DeviceIdType.MESH,
            )

        def fwd_q(direction, sem, origin):
            q_rdma(direction, sem,
                   wqf.at[:, pl.ds(origin * CHUNK, CHUNK)], origin).start()

        def fwd_o(direction, sem, origin):
            o_rdma(direction, sem,
                   wof.at[pl.ds(origin * CHUNK, CHUNK), :], origin).start()

        rows = lax.broadcasted_iota(jnp.int32, (SQ, SQ), 0)
        cols = lax.broadcasted_iota(jnp.int32, (SQ, SQ), 1)
        bias = jnp.where((cols // 64) <= (rows // 64), 0.0, -1e9).astype(
            jnp.float32
        )

        def compute_chunk(o):
            q2 = jnp.dot(
                x_ref[:, :], wqf[:, pl.ds(o * CHUNK, CHUNK)],
                preferred_element_type=jnp.float32,
            ).astype(jnp.bfloat16)
            ctxs = []
            for b in range(B_LOC):
                hctx = []
                for t in range(H_LOC):
                    h = o * H_LOC + t
                    qh = q2[b * SQ:(b + 1) * SQ, t * DH:(t + 1) * DH]
                    s = lax.dot_general(
                        qh, k_ref[b, h], (((1,), (1,)), ((), ())),
                        preferred_element_type=jnp.float32,
                    ) * 0.125 + bias
                    w = jnp.exp(s)
                    r = 1.0 / jnp.sum(w, axis=1, keepdims=True)
                    ctx = jnp.dot(
                        w.astype(jnp.bfloat16), v_ref[b, h],
                        preferred_element_type=jnp.float32,
                    ) * r
                    hctx.append(ctx.astype(jnp.bfloat16))
                ctxs.append(jnp.concatenate(hctx, axis=1))
            ctx2 = jnp.concatenate(ctxs, axis=0)
            out_ref[:, :] += jnp.dot(
                ctx2, wof[pl.ds(o * CHUNK, CHUNK), :],
                preferred_element_type=jnp.float32,
            )

        q_rdma("r", 0, wq_ref, me).start()
        q_rdma("l", N_RIGHT, wq_ref, me).start()
        o_rdma("r", 0, wo_ref, me).start()
        o_rdma("l", N_RIGHT, wo_ref, me).start()

        wqf[:, pl.ds(me * CHUNK, CHUNK)] = wq_ref[:, :]
        wof[pl.ds(me * CHUNK, CHUNK), :] = wo_ref[:, :]
        out_ref[:, :] = jnp.zeros((MROW, DM), jnp.float32)
        compute_chunk(me)

        def step(s, _):
            r_o = lax.rem(me + 2 * N_DEV - 1 - s, N_DEV)
            l_o = lax.rem(me + 1 + s, N_DEV)
            dst_q_r = wqf.at[:, pl.ds(r_o * CHUNK, CHUNK)]
            dst_o_r = wof.at[pl.ds(r_o * CHUNK, CHUNK), :]
            dst_q_l = wqf.at[:, pl.ds(l_o * CHUNK, CHUNK)]
            dst_o_l = wof.at[pl.ds(l_o * CHUNK, CHUNK), :]

            q_rdma("r", s, dst_q_r, r_o).wait_recv()

            @pl.when(s + 1 < N_RIGHT)
            def _():
                fwd_q("r", s + 1, r_o)

            o_rdma("r", s, dst_o_r, r_o).wait_recv()

            @pl.when(s + 1 < N_RIGHT)
            def _():
                fwd_o("r", s + 1, r_o)

            @pl.when(s < N_LEFT)
            def _():
                q_rdma("l", N_RIGHT + s, dst_q_l, l_o).wait_recv()

            @pl.when(s + 1 < N_LEFT)
            def _():
                fwd_q("l", N_RIGHT + s + 1, l_o)

            @pl.when(s < N_LEFT)
            def _():
                o_rdma("l", N_RIGHT + s, dst_o_l, l_o).wait_recv()

            @pl.when(s + 1 < N_LEFT)
            def _():
                fwd_o("l", N_RIGHT + s + 1, l_o)

            compute_chunk(r_o)

            @pl.when(s < N_LEFT)
            def _():
                compute_chunk(l_o)

            return 0

        lax.fori_loop(0, N_RIGHT, step, 0)

        for h in range(N_RIGHT):
            q_rdma("r", h, wq_ref, me).wait_send()
            o_rdma("r", h, wo_ref, me).wait_send()
        for h in range(N_LEFT):
            q_rdma("l", N_RIGHT + h, wq_ref, me).wait_send()
            o_rdma("l", N_RIGHT + h, wo_ref, me).wait_send()

    out2 = pl.pallas_call(
        body,
        out_shape=jax.ShapeDtypeStruct((MROW, DM), jnp.float32),
        in_specs=[pl.BlockSpec(memory_space=pltpu.VMEM)] * 5,
        out_specs=pl.BlockSpec(memory_space=pltpu.VMEM),
        scratch_shapes=[
            pltpu.VMEM((DM, HD), jnp.bfloat16),
            pltpu.VMEM((HD, DM), jnp.bfloat16),
            pltpu.SemaphoreType.DMA((N_DEV - 1,)),
            pltpu.SemaphoreType.DMA((N_DEV - 1,)),
            pltpu.SemaphoreType.DMA((N_DEV - 1,)),
            pltpu.SemaphoreType.DMA((N_DEV - 1,)),
        ],
        compiler_params=pltpu.CompilerParams(collective_id=0),
    )(x2, wq, k, v, wo)
    return out2.reshape(B_LOC, SQ, DM)


# device time: 75063 ns/iter; 2.6829x vs baseline; 1.0490x over previous
import jax
import jax.numpy as jnp
import numpy as np
from jax import lax
from jax.experimental import pallas as pl
from jax.experimental.pallas import tpu as pltpu

N_DEV = 16
B_LOC = 2
SQ = 256
HQ = 64
DH = 64
DM = 512
HD = HQ * DH
H_LOC = HQ // N_DEV
CHUNK = H_LOC * DH
MROW = B_LOC * SQ

N_RIGHT = 8
N_LEFT = 7

CYCLE = np.array([0, 1, 5, 4, 8, 9, 13, 12, 15, 14, 10, 11, 7, 6, 2, 3])
POS = np.argsort(CYCLE)


def kernel(x, Wq, K_ext, V_ext, Wo):
    my = lax.axis_index("i")

    x2 = x.astype(jnp.bfloat16).reshape(MROW, DM)
    wq = Wq.astype(jnp.bfloat16)
    wo = Wo.astype(jnp.bfloat16)
    k = lax.dynamic_slice_in_dim(K_ext, my * B_LOC, B_LOC, axis=0)
    v = lax.dynamic_slice_in_dim(V_ext, my * B_LOC, B_LOC, axis=0)
    k = k.transpose(0, 2, 1, 3).astype(jnp.bfloat16)
    v = v.transpose(0, 2, 1, 3).astype(jnp.bfloat16)

    cyc = jnp.asarray(CYCLE, jnp.int32)
    pos = jnp.asarray(POS, jnp.int32)[my]
    nbrs = jnp.stack([
        cyc[(pos + N_DEV - 1) % N_DEV],
        cyc[(pos + 1) % N_DEV],
    ]).astype(jnp.int32)
    r_origs = cyc[(pos + 2 * N_DEV - 1 - jnp.arange(N_RIGHT)) % N_DEV]
    l_origs = cyc[(pos + 1 + jnp.arange(N_LEFT)) % N_DEV]

    def body(x_ref, wq_ref, k_ref, v_ref, wo_ref,
             nbr_ref, ro_ref, lo_ref, out_ref,
             wqf, wof, sq_s, sq_r, so_s, so_r):
        me = lax.axis_index("i")
        left = nbr_ref[0]
        right = nbr_ref[1]

        bar = pltpu.get_barrier_semaphore()
        pl.semaphore_signal(bar, inc=1, device_id=(left,),
                            device_id_type=pl.DeviceIdType.MESH)
        pl.semaphore_signal(bar, inc=1, device_id=(right,),
                            device_id_type=pl.DeviceIdType.MESH)
        pl.semaphore_wait(bar, 2)

        def q_rdma(direction, sem, src, origin):
            tgt = right if direction == "r" else left
            return pltpu.make_async_remote_copy(
                src_ref=src,
                dst_ref=wqf.at[:, pl.ds(origin * CHUNK, CHUNK)],
                send_sem=sq_s.at[sem], recv_sem=sq_r.at[sem],
                device_id=(tgt,), device_id_type=pl.DeviceIdType.MESH,
            )

        def o_rdma(direction, sem, src, origin):
            tgt = right if direction == "r" else left
            return pltpu.make_async_remote_copy(
                src_ref=src,
                dst_ref=wof.at[pl.ds(origin * CHUNK, CHUNK), :],
                send_sem=so_s.at[sem], recv_sem=so_r.at[sem],
                device_id=(tgt,), device_id_type=pl.DeviceIdType.MESH,
            )

        def fwd_q(direction, sem, origin):
            q_rdma(direction, sem,
                   wqf.at[:, pl.ds(origin * CHUNK, CHUNK)], origin).start()

        def fwd_o(direction, sem, origin):
            o_rdma(direction, sem,
                   wof.at[pl.ds(origin * CHUNK, CHUNK), :], origin).start()

        rows = lax.broadcasted_iota(jnp.int32, (SQ, SQ), 0)
        cols = lax.broadcasted_iota(jnp.int32, (SQ, SQ), 1)
        bias = jnp.where((cols // 64) <= (rows // 64), 0.0, -1e9).astype(
            jnp.float32
        )

        def compute_chunk(o):
            q2 = jnp.dot(
                x_ref[:, :], wqf[:, pl.ds(o * CHUNK, CHUNK)],
                preferred_element_type=jnp.float32,
            ).astype(jnp.bfloat16)
            ctxs = []
            for b in range(B_LOC):
                hctx = []
                for t in range(H_LOC):
                    h = o * H_LOC + t
                    qh = q2[b * SQ:(b + 1) * SQ, t * DH:(t + 1) * DH]
                    s = lax.dot_general(
                        qh, k_ref[b, h], (((1,), (1,)), ((), ())),
                        preferred_element_type=jnp.float32,
                    ) * 0.125 + bias
                    w = jnp.exp(s)
                    r = 1.0 / jnp.sum(w, axis=1, keepdims=True)
                    ctx = jnp.dot(
                        w.astype(jnp.bfloat16), v_ref[b, h],
                        preferred_element_type=jnp.float32,
                    ) * r
                    hctx.append(ctx.astype(jnp.bfloat16))
                ctxs.append(jnp.concatenate(hctx, axis=1))
            ctx2 = jnp.concatenate(ctxs, axis=0)
            out_ref[:, :] += jnp.dot(
                ctx2, wof[pl.ds(o * CHUNK, CHUNK), :],
                preferred_element_type=jnp.float32,
            )

        q_rdma("r", 0, wq_ref, me).start()
        q_rdma("l", N_RIGHT, wq_ref, me).start()
        o_rdma("r", 0, wo_ref, me).start()
        o_rdma("l", N_RIGHT, wo_ref, me).start()

        wqf[:, pl.ds(me * CHUNK, CHUNK)] = wq_ref[:, :]
        wof[pl.ds(me * CHUNK, CHUNK), :] = wo_ref[:, :]
        out_ref[:, :] = jnp.zeros((MROW, DM), jnp.float32)
        compute_chunk(me)

        def step(s, _):
            r_o = ro_ref[s]
            l_o = lo_ref[jnp.minimum(s, N_LEFT - 1)]
            dst_q_r = wqf.at[:, pl.ds(r_o * CHUNK, CHUNK)]
            dst_o_r = wof.at[pl.ds(r_o * CHUNK, CHUNK), :]
            dst_q_l = wqf.at[:, pl.ds(l_o * CHUNK, CHUNK)]
            dst_o_l = wof.at[pl.ds(l_o * CHUNK, CHUNK), :]

            q_rdma("r", s, dst_q_r, r_o).wait_recv()

            @pl.when(s + 1 < N_RIGHT)
            def _():
                fwd_q("r", s + 1, r_o)

            o_rdma("r", s, dst_o_r, r_o).wait_recv()

            @pl.when(s + 1 < N_RIGHT)
            def _():
                fwd_o("r", s + 1, r_o)

            @pl.when(s < N_LEFT)
            def _():
                q_rdma("l", N_RIGHT + s, dst_q_l, l_o).wait_recv()

            @pl.when(s + 1 < N_LEFT)
            def _():
                fwd_q("l", N_RIGHT + s + 1, l_o)

            @pl.when(s < N_LEFT)
            def _():
                o_rdma("l", N_RIGHT + s, dst_o_l, l_o).wait_recv()

            @pl.when(s + 1 < N_LEFT)
            def _():
                fwd_o("l", N_RIGHT + s + 1, l_o)

            compute_chunk(r_o)

            @pl.when(s < N_LEFT)
            def _():
                compute_chunk(l_o)

            return 0

        lax.fori_loop(0, N_RIGHT, step, 0)

        for h in range(N_RIGHT):
            q_rdma("r", h, wq_ref, me).wait_send()
            o_rdma("r", h, wo_ref, me).wait_send()
        for h in range(N_LEFT):
            q_rdma("l", N_RIGHT + h, wq_ref, me).wait_send()
            o_rdma("l", N_RIGHT + h, wo_ref, me).wait_send()

    out2 = pl.pallas_call(
        body,
        out_shape=jax.ShapeDtypeStruct((MROW, DM), jnp.float32),
        in_specs=(
            [pl.BlockSpec(memory_space=pltpu.VMEM)] * 5
            + [pl.BlockSpec(memory_space=pltpu.SMEM)] * 3
        ),
        out_specs=pl.BlockSpec(memory_space=pltpu.VMEM),
        scratch_shapes=[
            pltpu.VMEM((DM, HD), jnp.bfloat16),
            pltpu.VMEM((HD, DM), jnp.bfloat16),
            pltpu.SemaphoreType.DMA((N_DEV - 1,)),
            pltpu.SemaphoreType.DMA((N_DEV - 1,)),
            pltpu.SemaphoreType.DMA((N_DEV - 1,)),
            pltpu.SemaphoreType.DMA((N_DEV - 1,)),
        ],
        compiler_params=pltpu.CompilerParams(collective_id=0),
    )(x2, wq, k, v, wo, nbrs,
      r_origs.astype(jnp.int32), l_origs.astype(jnp.int32))
    return out2.reshape(B_LOC, SQ, DM)
